# Initial kernel scaffold; baseline (speedup 1.0000x reference)
import jax
import jax.numpy as jnp
from jax.experimental import pallas as pl


def kernel(pseudo_latent, pseudo_labels, true_latent, true_labels, centroids_pseudo, pseudo_count, centroids_true, true_count, ncells):
    raise NotImplementedError("write your pallas kernel here")


# speedup vs baseline: 5.0667x; 5.0667x over previous
import functools

import jax
import jax.numpy as jnp
from jax.experimental import pallas as pl
from jax.experimental.pallas import tpu as pltpu

N_CELLS = 16384
N_HIDDEN = 256
N_LABELS = 512
BLK = 1024
NBLK = N_CELLS // BLK


def _seg_loss_kernel(pl_lat, pl_lab, tr_lat, tr_lab, cp_t, ct_t, pc, tc, nc,
                     out, sums_p, cnt_p, sums_t, cnt_t):
    i = pl.program_id(0)

    @pl.when(i == 0)
    def _init():
        sums_p[...] = jnp.zeros_like(sums_p)
        cnt_p[...] = jnp.zeros_like(cnt_p)
        sums_t[...] = jnp.zeros_like(sums_t)
        cnt_t[...] = jnp.zeros_like(cnt_t)

    lane_ids = jax.lax.broadcasted_iota(jnp.int32, (BLK, N_LABELS), 1)
    ones_col = jnp.ones((BLK, 1), dtype=jnp.float32)
    dn = (((0,), (0,)), ((), ()))

    oh_p = (pl_lab[0] == lane_ids).astype(jnp.float32)
    sums_p[...] += jax.lax.dot_general(oh_p, pl_lat[...], dn,
                                       preferred_element_type=jnp.float32)
    cnt_p[...] += jax.lax.dot_general(oh_p, ones_col, dn,
                                      preferred_element_type=jnp.float32)

    oh_t = (tr_lab[0] == lane_ids).astype(jnp.float32)
    sums_t[...] += jax.lax.dot_general(oh_t, tr_lat[...], dn,
                                       preferred_element_type=jnp.float32)
    cnt_t[...] += jax.lax.dot_general(oh_t, ones_col, dn,
                                      preferred_element_type=jnp.float32)

    @pl.when(i == NBLK - 1)
    def _combine():
        counts_p = cnt_p[...]
        counts_t = cnt_t[...]
        reset = jnp.max(pc[...]) >= float(N_LABELS) * 1000.0
        pcs = jnp.where(reset, jnp.ones_like(pc[...]), pc[...])
        tcs = tc[...]

        cent_p = sums_p[...] / jnp.maximum(counts_p, 1.0)
        mask_p = counts_p > 5.0
        cp_new = jnp.where(mask_p,
                           (cp_t[...] * pcs + cent_p * counts_p) / (pcs + counts_p),
                           cp_t[...])

        cent_t = sums_t[...] / jnp.maximum(counts_t, 1.0)
        mask_t = counts_t > 5.0
        ct_new = jnp.where(mask_t,
                           (ct_t[...] * tcs + cent_t * counts_t) / (tcs + counts_t),
                           ct_t[...])

        mse = jnp.sum((cp_new - ct_new) ** 2) / float(N_LABELS * N_HIDDEN)

        pc_new = jnp.where(mask_p, pcs + counts_p, pcs)
        t = pc_new / jnp.sum(pc_new)
        kl = jnp.sum(jnp.where(t > 0.0,
                               t * (jnp.log(t) - jnp.log(nc[...])),
                               0.0)) / float(N_LABELS)
        out[...] = jnp.reshape(mse + kl, (1, 1))


@jax.jit
def _run(pl_lat, pl_lab_r, tr_lat, tr_lab_r, cp_t, ct_t, pc, tc, nc):
    f32 = jnp.float32
    out = pl.pallas_call(
        _seg_loss_kernel,
        grid=(NBLK,),
        in_specs=[
            pl.BlockSpec((BLK, N_HIDDEN), lambda i: (i, 0)),
            pl.BlockSpec((1, BLK, 1), lambda i: (i, 0, 0)),
            pl.BlockSpec((BLK, N_HIDDEN), lambda i: (i, 0)),
            pl.BlockSpec((1, BLK, 1), lambda i: (i, 0, 0)),
            pl.BlockSpec((N_LABELS, N_HIDDEN), lambda i: (0, 0)),
            pl.BlockSpec((N_LABELS, N_HIDDEN), lambda i: (0, 0)),
            pl.BlockSpec((N_LABELS, 1), lambda i: (0, 0)),
            pl.BlockSpec((N_LABELS, 1), lambda i: (0, 0)),
            pl.BlockSpec((N_LABELS, 1), lambda i: (0, 0)),
        ],
        out_specs=pl.BlockSpec((1, 1), lambda i: (0, 0)),
        out_shape=jax.ShapeDtypeStruct((1, 1), f32),
        scratch_shapes=[
            pltpu.VMEM((N_LABELS, N_HIDDEN), f32),
            pltpu.VMEM((N_LABELS, 1), f32),
            pltpu.VMEM((N_LABELS, N_HIDDEN), f32),
            pltpu.VMEM((N_LABELS, 1), f32),
        ],
    )(pl_lat, pl_lab_r, tr_lat, tr_lab_r, cp_t, ct_t, pc, tc, nc)
    return out[0, 0]


def kernel(pseudo_latent, pseudo_labels, true_latent, true_labels,
           centroids_pseudo, pseudo_count, centroids_true, true_count, ncells):
    pl_lab_r = pseudo_labels.astype(jnp.int32).reshape(NBLK, BLK, 1)
    tr_lab_r = true_labels.astype(jnp.int32).reshape(NBLK, BLK, 1)
    cp_t = centroids_pseudo.T
    ct_t = centroids_true.T
    pc = pseudo_count.reshape(N_LABELS, 1)
    tc = true_count.reshape(N_LABELS, 1)
    nc = ncells.reshape(N_LABELS, 1)
    return _run(pseudo_latent, pl_lab_r, true_latent, tr_lab_r,
                cp_t, ct_t, pc, tc, nc)
